# baseline (speedup 1.0000x reference)
import numpy as np
import jax
import jax.numpy as jnp
from jax.experimental import pallas as pl
from jax.experimental.pallas import tpu as pltpu

V = 1_000_000
ROWS = 32
LANES = 128
BLOCK_C = 8192
CHUNKS = BLOCK_C // LANES
NB = (V + BLOCK_C - 1) // BLOCK_C
TAIL_COLS = V - (NB - 1) * BLOCK_C
TAIL_CHUNKS = (TAIL_COLS + LANES - 1) // LANES

TINY = np.float32(np.finfo(np.float32).tiny)
NEG_INF = np.float32(-np.inf)
INT_MAX = np.int32(np.iinfo(np.int32).max)
TEMP_EPS = np.float32(1e-6)

_KS0 = np.uint32(0)
_KS1 = np.uint32(42)
_KS2 = np.uint32(np.uint32(0) ^ np.uint32(42) ^ np.uint32(0x1BD11BDA))
_KS = (_KS0, _KS1, _KS2)
_ROT_A = (13, 15, 26, 6)
_ROT_B = (17, 29, 16, 24)
_ROUNDS = (_ROT_A, _ROT_B, _ROT_A, _ROT_B, _ROT_A)
_INJECT = ((1, 2, 1), (2, 0, 2), (0, 1, 3), (1, 2, 4), (2, 0, 5))


def _threefry_bits_from_x1(x1):
    x0 = x1
    r = _ROT_A[0]
    x1 = x0 ^ ((x1 << np.uint32(r)) | (x1 >> np.uint32(32 - r)))
    first = True
    for rots, (a, b, c) in zip(_ROUNDS, _INJECT):
        for r in rots[1:] if first else rots:
            x0 = x0 + x1
            x1 = (x1 << np.uint32(r)) | (x1 >> np.uint32(32 - r))
            x1 = x0 ^ x1
        first = False
        if _KS[a]:
            x0 = x0 + _KS[a]
        x1 = x1 + np.uint32(_KS[b] + np.uint32(c))
    return x0 ^ x1


def _uniform_from_bits(bits):
    fb = (bits >> np.uint32(9)) | np.uint32(0x3F800000)
    floats = jax.lax.bitcast_convert_type(fb, jnp.float32) - np.float32(1.0)
    return jnp.maximum(TINY, floats)


def _body(logits_ref, temps_ref, out_ref, lm, vu, vi, vx, rt):
    j = pl.program_id(0)
    lane = jax.lax.broadcasted_iota(jnp.int32, (ROWS, LANES), 1)
    rowbase = jax.lax.broadcasted_iota(jnp.int32, (ROWS, LANES), 0) * V

    @pl.when(j == 0)
    def _init():
        lm[...] = jnp.full((ROWS, LANES), NEG_INF, jnp.float32)
        vu[...] = jnp.full((ROWS, LANES), NEG_INF, jnp.float32)
        vi[...] = jnp.zeros((ROWS, LANES), jnp.int32)
        vx[...] = jnp.zeros((ROWS, LANES), jnp.float32)
        tb = jnp.broadcast_to(temps_ref[...], (ROWS, LANES))
        rt[...] = jnp.where(
            tb <= TEMP_EPS, np.float32(1e30), np.float32(1.0) / tb)

    def _scan_chunks(nchunks, mask_last):
        rtv = rt[...]
        rowlane42 = (rowbase + lane) + np.int32(42)
        base = j * BLOCK_C
        race = [None, None]
        lmax = [lm[...], None]
        for k in range(nchunks):
            x = logits_ref[:, k * LANES:(k + 1) * LANES]
            x1 = rowlane42 + (base + (k * LANES))
            bits = _threefry_bits_from_x1(x1.astype(jnp.uint32))
            u01 = _uniform_from_bits(bits)
            u = x * rtv - jnp.log(-jnp.log(u01))
            if mask_last and k == nchunks - 1:
                valid = lane < np.int32(((V - 1) % LANES) + 1)
                x = jnp.where(valid, x, NEG_INF)
                u = jnp.where(valid, u, NEG_INF)
            p = k & 1
            lmax[p] = x if lmax[p] is None else jnp.maximum(lmax[p], x)
            if race[p] is None:
                race[p] = (u, x1, x)
            else:
                bu, bi, bx = race[p]
                m = u > bu
                race[p] = (jnp.where(m, u, bu), jnp.where(m, x1, bi),
                           jnp.where(m, x, bx))
        if race[1] is None:
            bu, bi, bx = race[0]
            lmerged = lmax[0]
        else:
            u0, i0, x0 = race[0]
            u1, i1, x1_ = race[1]
            m = u1 > u0
            bu = jnp.where(m, u1, u0)
            bi = jnp.where(m, i1, i0)
            bx = jnp.where(m, x1_, x0)
            lmerged = jnp.maximum(lmax[0], lmax[1])
        lm[...] = lmerged
        w = vu[...]
        m = bu > w
        vu[...] = jnp.where(m, bu, w)
        vi[...] = jnp.where(m, bi, vi[...])
        vx[...] = jnp.where(m, bx, vx[...])

    @pl.when(j != NB - 1)
    def _main():
        _scan_chunks(CHUNKS, False)

    @pl.when(j == NB - 1)
    def _tail():
        _scan_chunks(TAIL_CHUNKS, TAIL_COLS % LANES != 0)

        tb = jnp.broadcast_to(temps_ref[...], (ROWS, LANES))
        st = jnp.where(tb <= TEMP_EPS, np.float32(1.0), tb)
        m_row = jnp.max(lm[...], axis=1, keepdims=True)
        cand_x1 = vi[...]
        col = (cand_x1 - np.int32(42)) - rowbase
        g = -jnp.log(-jnp.log(
            _uniform_from_bits(_threefry_bits_from_x1(
                cand_x1.astype(jnp.uint32)))))
        big_m = m_row / st[:, :1]
        b = (vx[...] / st - big_m) + g
        b_row = jnp.max(b, axis=1, keepdims=True)
        scol = jnp.min(jnp.where(b == b_row, col, INT_MAX),
                       axis=1, keepdims=True)
        cand_l = vx[...]
        gcol = jnp.min(jnp.where(cand_l == m_row, col, INT_MAX),
                       axis=1, keepdims=True)
        t = temps_ref[...]
        out_ref[...] = jnp.where(t <= TEMP_EPS, gcol, scol)


def kernel(logits, temperatures):
    temps2d = temperatures.reshape(ROWS, 1)
    out = pl.pallas_call(
        _body,
        grid=(NB,),
        in_specs=[
            pl.BlockSpec((ROWS, BLOCK_C), lambda j: (0, j)),
            pl.BlockSpec((ROWS, 1), lambda j: (0, 0)),
        ],
        out_specs=pl.BlockSpec((ROWS, 1), lambda j: (0, 0)),
        out_shape=jax.ShapeDtypeStruct((ROWS, 1), jnp.int32),
        scratch_shapes=[
            pltpu.VMEM((ROWS, LANES), jnp.float32),
            pltpu.VMEM((ROWS, LANES), jnp.float32),
            pltpu.VMEM((ROWS, LANES), jnp.int32),
            pltpu.VMEM((ROWS, LANES), jnp.float32),
            pltpu.VMEM((ROWS, LANES), jnp.float32),
        ],
    )(logits, temps2d)
    return out.reshape(ROWS)


# speedup vs baseline: 1.0152x; 1.0152x over previous
import numpy as np
import jax
import jax.numpy as jnp
from jax.experimental import pallas as pl
from jax.experimental.pallas import tpu as pltpu

V = 1_000_000
ROWS = 32
LANES = 128
BLOCK_C = 8192
CHUNKS = BLOCK_C // LANES
NB = (V + BLOCK_C - 1) // BLOCK_C
TAIL_COLS = V - (NB - 1) * BLOCK_C
TAIL_CHUNKS = (TAIL_COLS + LANES - 1) // LANES

TINY = np.float32(np.finfo(np.float32).tiny)
NEG_INF = np.float32(-np.inf)
INT_MAX = np.int32(np.iinfo(np.int32).max)
TEMP_EPS = np.float32(1e-6)

_KS0 = np.uint32(0)
_KS1 = np.uint32(42)
_KS2 = np.uint32(np.uint32(0) ^ np.uint32(42) ^ np.uint32(0x1BD11BDA))
_KS = (_KS0, _KS1, _KS2)
_ROT_A = (13, 15, 26, 6)
_ROT_B = (17, 29, 16, 24)
_ROUNDS = (_ROT_A, _ROT_B, _ROT_A, _ROT_B, _ROT_A)
_INJECT = ((1, 2, 1), (2, 0, 2), (0, 1, 3), (1, 2, 4), (2, 0, 5))


def _threefry_bits_from_x1(x1):
    x0 = x1
    r = _ROT_A[0]
    x1 = x0 ^ ((x1 << np.uint32(r)) | (x1 >> np.uint32(32 - r)))
    first = True
    for rots, (a, b, c) in zip(_ROUNDS, _INJECT):
        for r in rots[1:] if first else rots:
            x0 = x0 + x1
            x1 = (x1 << np.uint32(r)) | (x1 >> np.uint32(32 - r))
            x1 = x0 ^ x1
        first = False
        if _KS[a]:
            x0 = x0 + _KS[a]
        x1 = x1 + np.uint32(_KS[b] + np.uint32(c))
    return x0 ^ x1


def _uniform_from_bits(bits):
    fb = (bits >> np.uint32(9)) | np.uint32(0x3F800000)
    floats = jax.lax.bitcast_convert_type(fb, jnp.float32) - np.float32(1.0)
    return jnp.maximum(TINY, floats)


def _body(logits_ref, temps_ref, out_ref, lm, vu, vi, vx, rt):
    j = pl.program_id(0)
    lane = jax.lax.broadcasted_iota(jnp.int32, (ROWS, LANES), 1)
    rowbase = jax.lax.broadcasted_iota(jnp.int32, (ROWS, LANES), 0) * V

    @pl.when(j == 0)
    def _init():
        lm[...] = jnp.full((ROWS, LANES), NEG_INF, jnp.float32)
        vu[...] = jnp.full((ROWS, LANES), NEG_INF, jnp.float32)
        vi[...] = jnp.zeros((ROWS, LANES), jnp.int32)
        vx[...] = jnp.zeros((ROWS, LANES), jnp.float32)
        tb = jnp.broadcast_to(temps_ref[...], (ROWS, LANES))
        rt[...] = jnp.where(
            tb <= TEMP_EPS, np.float32(1e30), np.float32(1.0) / tb)

    def _scan_chunks(nchunks, mask_last):
        rtv = rt[...]
        rowlane42 = (rowbase + lane) + np.int32(42)
        base = j * BLOCK_C
        bu = bi = bx = None
        lmax = lm[...]
        for k in range(nchunks):
            x = logits_ref[:, k * LANES:(k + 1) * LANES]
            x1 = rowlane42 + (base + (k * LANES))
            bits = _threefry_bits_from_x1(x1.astype(jnp.uint32))
            fb = (bits >> np.uint32(9)) | np.uint32(0x3F800000)
            floats = jax.lax.bitcast_convert_type(
                fb, jnp.float32) - np.float32(1.0)
            u = x * rtv - jnp.log(-jnp.log(floats))
            if mask_last and k == nchunks - 1:
                valid = lane < np.int32(((V - 1) % LANES) + 1)
                x = jnp.where(valid, x, NEG_INF)
                u = jnp.where(valid, u, NEG_INF)
            lmax = jnp.maximum(lmax, x)
            if bu is None:
                bu, bi, bx = u, x1, x
            else:
                m = u > bu
                bu = jnp.where(m, u, bu)
                bi = jnp.where(m, x1, bi)
                bx = jnp.where(m, x, bx)
        lm[...] = lmax
        w = vu[...]
        m = bu > w
        vu[...] = jnp.where(m, bu, w)
        vi[...] = jnp.where(m, bi, vi[...])
        vx[...] = jnp.where(m, bx, vx[...])

    @pl.when(j != NB - 1)
    def _main():
        _scan_chunks(CHUNKS, False)

    @pl.when(j == NB - 1)
    def _tail():
        _scan_chunks(TAIL_CHUNKS, TAIL_COLS % LANES != 0)

        tb = jnp.broadcast_to(temps_ref[...], (ROWS, LANES))
        st = jnp.where(tb <= TEMP_EPS, np.float32(1.0), tb)
        m_row = jnp.max(lm[...], axis=1, keepdims=True)
        cand_x1 = vi[...]
        col = (cand_x1 - np.int32(42)) - rowbase
        g = -jnp.log(-jnp.log(
            _uniform_from_bits(_threefry_bits_from_x1(
                cand_x1.astype(jnp.uint32)))))
        big_m = m_row / st[:, :1]
        b = (vx[...] / st - big_m) + g
        b_row = jnp.max(b, axis=1, keepdims=True)
        scol = jnp.min(jnp.where(b == b_row, col, INT_MAX),
                       axis=1, keepdims=True)
        cand_l = vx[...]
        gcol = jnp.min(jnp.where(cand_l == m_row, col, INT_MAX),
                       axis=1, keepdims=True)
        t = temps_ref[...]
        out_ref[...] = jnp.where(t <= TEMP_EPS, gcol, scol)


def kernel(logits, temperatures):
    temps2d = temperatures.reshape(ROWS, 1)
    out = pl.pallas_call(
        _body,
        grid=(NB,),
        in_specs=[
            pl.BlockSpec((ROWS, BLOCK_C), lambda j: (0, j)),
            pl.BlockSpec((ROWS, 1), lambda j: (0, 0)),
        ],
        out_specs=pl.BlockSpec((ROWS, 1), lambda j: (0, 0)),
        out_shape=jax.ShapeDtypeStruct((ROWS, 1), jnp.int32),
        scratch_shapes=[
            pltpu.VMEM((ROWS, LANES), jnp.float32),
            pltpu.VMEM((ROWS, LANES), jnp.float32),
            pltpu.VMEM((ROWS, LANES), jnp.int32),
            pltpu.VMEM((ROWS, LANES), jnp.float32),
            pltpu.VMEM((ROWS, LANES), jnp.float32),
        ],
    )(logits, temps2d)
    return out.reshape(ROWS)


# speedup vs baseline: 1.0214x; 1.0061x over previous
import numpy as np
import jax
import jax.numpy as jnp
from jax.experimental import pallas as pl
from jax.experimental.pallas import tpu as pltpu

V = 1_000_000
ROWS = 32
LANES = 128
BLOCK_C = 16384
CHUNKS = BLOCK_C // LANES
NB = (V + BLOCK_C - 1) // BLOCK_C
TAIL_COLS = V - (NB - 1) * BLOCK_C
TAIL_CHUNKS = (TAIL_COLS + LANES - 1) // LANES

TINY = np.float32(np.finfo(np.float32).tiny)
NEG_INF = np.float32(-np.inf)
INT_MAX = np.int32(np.iinfo(np.int32).max)
TEMP_EPS = np.float32(1e-6)

_KS0 = np.uint32(0)
_KS1 = np.uint32(42)
_KS2 = np.uint32(np.uint32(0) ^ np.uint32(42) ^ np.uint32(0x1BD11BDA))
_KS = (_KS0, _KS1, _KS2)
_ROT_A = (13, 15, 26, 6)
_ROT_B = (17, 29, 16, 24)
_ROUNDS = (_ROT_A, _ROT_B, _ROT_A, _ROT_B, _ROT_A)
_INJECT = ((1, 2, 1), (2, 0, 2), (0, 1, 3), (1, 2, 4), (2, 0, 5))


def _threefry_bits_from_x1(x1):
    x0 = x1
    r = _ROT_A[0]
    x1 = x0 ^ ((x1 << np.uint32(r)) | (x1 >> np.uint32(32 - r)))
    first = True
    for rots, (a, b, c) in zip(_ROUNDS, _INJECT):
        for r in rots[1:] if first else rots:
            x0 = x0 + x1
            x1 = (x1 << np.uint32(r)) | (x1 >> np.uint32(32 - r))
            x1 = x0 ^ x1
        first = False
        if _KS[a]:
            x0 = x0 + _KS[a]
        x1 = x1 + np.uint32(_KS[b] + np.uint32(c))
    return x0 ^ x1


def _uniform_from_bits(bits):
    fb = (bits >> np.uint32(9)) | np.uint32(0x3F800000)
    floats = jax.lax.bitcast_convert_type(fb, jnp.float32) - np.float32(1.0)
    return jnp.maximum(TINY, floats)


def _body(logits_ref, temps_ref, out_ref, lm, vu, vi, vx, rt):
    j = pl.program_id(0)
    lane = jax.lax.broadcasted_iota(jnp.int32, (ROWS, LANES), 1)
    rowbase = jax.lax.broadcasted_iota(jnp.int32, (ROWS, LANES), 0) * V

    @pl.when(j == 0)
    def _init():
        lm[...] = jnp.full((ROWS, LANES), NEG_INF, jnp.float32)
        vu[...] = jnp.full((ROWS, LANES), NEG_INF, jnp.float32)
        vi[...] = jnp.zeros((ROWS, LANES), jnp.int32)
        vx[...] = jnp.zeros((ROWS, LANES), jnp.float32)
        tb = jnp.broadcast_to(temps_ref[...], (ROWS, LANES))
        rt[...] = jnp.where(
            tb <= TEMP_EPS, np.float32(1e30), np.float32(1.0) / tb)

    def _scan_chunks(nchunks, mask_last):
        rtv = rt[...]
        rowlane42 = (rowbase + lane) + np.int32(42)
        base = j * BLOCK_C
        bu = bi = bx = None
        lmax = lm[...]
        for k in range(nchunks):
            x = logits_ref[:, k * LANES:(k + 1) * LANES]
            x1 = rowlane42 + (base + (k * LANES))
            bits = _threefry_bits_from_x1(x1.astype(jnp.uint32))
            fb = (bits >> np.uint32(9)) | np.uint32(0x3F800000)
            floats = jax.lax.bitcast_convert_type(
                fb, jnp.float32) - np.float32(1.0)
            u = x * rtv - jnp.log(-jnp.log(floats))
            if mask_last and k == nchunks - 1:
                valid = lane < np.int32(((V - 1) % LANES) + 1)
                x = jnp.where(valid, x, NEG_INF)
                u = jnp.where(valid, u, NEG_INF)
            lmax = jnp.maximum(lmax, x)
            if bu is None:
                bu, bi, bx = u, x1, x
            else:
                m = u > bu
                bu = jnp.where(m, u, bu)
                bi = jnp.where(m, x1, bi)
                bx = jnp.where(m, x, bx)
        lm[...] = lmax
        w = vu[...]
        m = bu > w
        vu[...] = jnp.where(m, bu, w)
        vi[...] = jnp.where(m, bi, vi[...])
        vx[...] = jnp.where(m, bx, vx[...])

    @pl.when(j != NB - 1)
    def _main():
        _scan_chunks(CHUNKS, False)

    @pl.when(j == NB - 1)
    def _tail():
        _scan_chunks(TAIL_CHUNKS, TAIL_COLS % LANES != 0)

        tb = jnp.broadcast_to(temps_ref[...], (ROWS, LANES))
        st = jnp.where(tb <= TEMP_EPS, np.float32(1.0), tb)
        m_row = jnp.max(lm[...], axis=1, keepdims=True)
        cand_x1 = vi[...]
        col = (cand_x1 - np.int32(42)) - rowbase
        g = -jnp.log(-jnp.log(
            _uniform_from_bits(_threefry_bits_from_x1(
                cand_x1.astype(jnp.uint32)))))
        big_m = m_row / st[:, :1]
        b = (vx[...] / st - big_m) + g
        b_row = jnp.max(b, axis=1, keepdims=True)
        scol = jnp.min(jnp.where(b == b_row, col, INT_MAX),
                       axis=1, keepdims=True)
        cand_l = vx[...]
        gcol = jnp.min(jnp.where(cand_l == m_row, col, INT_MAX),
                       axis=1, keepdims=True)
        t = temps_ref[...]
        out_ref[...] = jnp.where(t <= TEMP_EPS, gcol, scol)


def kernel(logits, temperatures):
    temps2d = temperatures.reshape(ROWS, 1)
    out = pl.pallas_call(
        _body,
        grid=(NB,),
        in_specs=[
            pl.BlockSpec((ROWS, BLOCK_C), lambda j: (0, j)),
            pl.BlockSpec((ROWS, 1), lambda j: (0, 0)),
        ],
        out_specs=pl.BlockSpec((ROWS, 1), lambda j: (0, 0)),
        out_shape=jax.ShapeDtypeStruct((ROWS, 1), jnp.int32),
        scratch_shapes=[
            pltpu.VMEM((ROWS, LANES), jnp.float32),
            pltpu.VMEM((ROWS, LANES), jnp.float32),
            pltpu.VMEM((ROWS, LANES), jnp.int32),
            pltpu.VMEM((ROWS, LANES), jnp.float32),
            pltpu.VMEM((ROWS, LANES), jnp.float32),
        ],
    )(logits, temps2d)
    return out.reshape(ROWS)
